# Initial kernel scaffold; baseline (speedup 1.0000x reference)
import jax
import jax.numpy as jnp
from jax.experimental import pallas as pl


def kernel(x, W, b, gumbel_u):
    raise NotImplementedError("write your pallas kernel here")


# speedup vs baseline: 1.2972x; 1.2972x over previous
import jax
import jax.numpy as jnp
from jax.experimental import pallas as pl

N = 16384
D = 1024
BLK = 1024


def _body(x_ref, w_ref, b_ref, u_ref, o_ref):
    x = x_ref[...]
    logits = jax.lax.dot_general(
        x, w_ref[...],
        dimension_numbers=(((1,), (1,)), ((), ())),
        precision=jax.lax.Precision.DEFAULT,
        preferred_element_type=jnp.float32,
    ) + b_ref[...]
    u = u_ref[...]
    g = -jnp.log(-jnp.log(u + 1e-10) + 1e-10)
    z = (logits + g) / 0.5
    mask = (z[:, 1] > z[:, 0]).astype(x.dtype)
    o_ref[...] = x * mask[:, None]


def kernel(x, W, b, gumbel_u):
    b2 = b.reshape(1, 2)
    return pl.pallas_call(
        _body,
        grid=(N // BLK,),
        in_specs=[
            pl.BlockSpec((BLK, D), lambda i: (i, 0)),
            pl.BlockSpec((2, D), lambda i: (0, 0)),
            pl.BlockSpec((1, 2), lambda i: (0, 0)),
            pl.BlockSpec((BLK, 2), lambda i: (i, 0)),
        ],
        out_specs=pl.BlockSpec((BLK, D), lambda i: (i, 0)),
        out_shape=jax.ShapeDtypeStruct((N, D), x.dtype),
    )(x, W, b2, gumbel_u)
